# speedup vs baseline: 2.7946x; 2.7946x over previous
import functools

import jax
import jax.numpy as jnp
from jax import lax
from jax.experimental import pallas as pl
from jax.experimental.pallas import tpu as pltpu
from jax.experimental.pallas import tpu_sc as plsc

BATCH = 16384
EMBED_DIM = 128
_CHUNK = 128


def _gather_tec_body(nc, bpw, uidx, iidx, utab, itab, ue_out, ie_out,
                     idx_v, rows_v, sem):
    wid = lax.axis_index("s") * nc + lax.axis_index("c")
    base = wid * bpw
    nchunks = bpw // _CHUNK
    pltpu.sync_copy(uidx.at[pl.ds(base, bpw)], idx_v)
    copies = [
        pltpu.async_copy(
            utab.at[idx_v.at[pl.ds(j * _CHUNK, _CHUNK)]],
            rows_v.at[pl.ds(j * _CHUNK, _CHUNK)],
            sem,
        )
        for j in range(nchunks)
    ]
    for c in copies:
        c.wait()
    pltpu.sync_copy(rows_v, ue_out.at[pl.ds(base, bpw)])
    pltpu.sync_copy(iidx.at[pl.ds(base, bpw)], idx_v)
    copies = [
        pltpu.async_copy(
            itab.at[idx_v.at[pl.ds(j * _CHUNK, _CHUNK)]],
            rows_v.at[pl.ds(j * _CHUNK, _CHUNK)],
            sem,
        )
        for j in range(nchunks)
    ]
    for c in copies:
        c.wait()
    pltpu.sync_copy(rows_v, ie_out.at[pl.ds(base, bpw)])


def _sc_gather(user_indices, item_indices, user_emb, item_emb):
    info = plsc.get_sparse_core_info()
    nc, ns = info.num_cores, info.num_subcores
    nw = nc * ns
    bpw = BATCH // nw
    mesh = plsc.VectorSubcoreMesh(core_axis_name="c", subcore_axis_name="s")
    k = pl.kernel(
        functools.partial(_gather_tec_body, nc, bpw),
        mesh=mesh,
        out_type=[
            jax.ShapeDtypeStruct((BATCH, EMBED_DIM), jnp.float32),
            jax.ShapeDtypeStruct((BATCH, EMBED_DIM), jnp.float32),
        ],
        scratch_types=[
            pltpu.VMEM((bpw,), jnp.int32),
            pltpu.VMEM((bpw, EMBED_DIM), jnp.float32),
            pltpu.SemaphoreType.DMA,
        ],
    )
    return k(user_indices, item_indices, user_emb, item_emb)


def _mlp_body(ue, ie, w1u, w1i, b1, w2, b2, w3, b3, wo, bo, out):
    x = jnp.dot(ue[...], w1u[...], preferred_element_type=jnp.float32)
    x = x + jnp.dot(ie[...], w1i[...], preferred_element_type=jnp.float32)
    x = jnp.maximum(x + b1[...], 0.0)
    x = jnp.maximum(jnp.dot(x, w2[...], preferred_element_type=jnp.float32) + b2[...], 0.0)
    x = jnp.maximum(jnp.dot(x, w3[...], preferred_element_type=jnp.float32) + b3[...], 0.0)
    out[...] = jnp.sum(x * wo[...], axis=1, keepdims=True) + bo[...]


def _tc_mlp(ue, ie, w1u_t, w1i_t, b1, w2_t, b2, w3_t, b3, wo, bo):
    blk = 2048
    grid = BATCH // blk
    full = lambda shape: pl.BlockSpec(shape, lambda i: (0, 0))
    out2d = pl.pallas_call(
        _mlp_body,
        grid=(grid,),
        in_specs=[
            pl.BlockSpec((blk, EMBED_DIM), lambda i: (i, 0)),
            pl.BlockSpec((blk, EMBED_DIM), lambda i: (i, 0)),
            full(w1u_t.shape),
            full(w1i_t.shape),
            full(b1.shape),
            full(w2_t.shape),
            full(b2.shape),
            full(w3_t.shape),
            full(b3.shape),
            full(wo.shape),
            full(bo.shape),
        ],
        out_specs=pl.BlockSpec((blk, 1), lambda i: (i, 0)),
        out_shape=jax.ShapeDtypeStruct((BATCH, 1), jnp.float32),
    )(ue, ie, w1u_t, w1i_t, b1, w2_t, b2, w3_t, b3, wo, bo)
    return out2d[:, 0]


def kernel(user_indices, item_indices, user_emb, item_emb,
           W1, b1, W2, b2, W3, b3, Wo, bo):
    user_indices = user_indices.astype(jnp.int32)
    item_indices = item_indices.astype(jnp.int32)
    ue, ie = _sc_gather(user_indices, item_indices, user_emb, item_emb)
    w1u_t = W1[:, :EMBED_DIM].T
    w1i_t = W1[:, EMBED_DIM:].T
    return _tc_mlp(
        ue, ie,
        w1u_t, w1i_t, b1.reshape(1, -1),
        W2.T, b2.reshape(1, -1),
        W3.T, b3.reshape(1, -1),
        Wo, bo.reshape(1, 1),
    )
